# Initial kernel scaffold; baseline (speedup 1.0000x reference)
import jax
import jax.numpy as jnp
from jax.experimental import pallas as pl


def kernel(x, edge_index, W1, b1, gamma1, beta1, W2, b2, gamma2, beta2, Wp1, bp1, Wp2, bp2, Wp3, bp3):
    raise NotImplementedError("write your pallas kernel here")


# speedup vs baseline: 7.4074x; 7.4074x over previous
import functools

import jax
import jax.numpy as jnp
from jax import lax
from jax.experimental import pallas as pl
from jax.experimental.pallas import tpu as pltpu
from jax.experimental.pallas import tpu_sc as plsc

N = 10000
E = 320000
NC = 2
NS = 16
NW = NC * NS
EPW = E // NW
CH = 125
NCH = EPW // CH
ROWS_PER_TILE = N // NS


def _seg_sum_sc(width):
    mesh = plsc.VectorSubcoreMesh(core_axis_name="c", subcore_axis_name="s")

    @functools.partial(
        pl.kernel,
        out_type=jax.ShapeDtypeStruct((NC, N, width), jnp.float32),
        mesh=mesh,
        compiler_params=pltpu.CompilerParams(use_tc_tiling_on_sc=False),
        scratch_types=[
            pltpu.VMEM((NCH, CH), jnp.int32),
            pltpu.VMEM((NCH, CH), jnp.int32),
            pltpu.VMEM((CH, width), jnp.float32),
            pltpu.VMEM_SHARED((N, width), jnp.float32),
            pltpu.SemaphoreType.DMA,
        ],
    )
    def k(table_hbm, src_hbm, dst_hbm, zeros_hbm, out_hbm,
          src_v, dst_v, rows_v, acc_sh, sem):
        c = lax.axis_index("c")
        s = lax.axis_index("s")
        wid = c * NS + s
        stripe = pl.ds(s * ROWS_PER_TILE, ROWS_PER_TILE)
        pltpu.sync_copy(zeros_hbm.at[stripe], acc_sh.at[stripe])
        pltpu.sync_copy(src_hbm.at[wid], src_v)
        pltpu.sync_copy(dst_hbm.at[wid], dst_v)
        plsc.subcore_barrier()

        def body(i, carry):
            pltpu.async_copy(table_hbm.at[src_v.at[i]], rows_v, sem).wait()
            pltpu.sync_copy(rows_v, acc_sh.at[dst_v.at[i]], add=True)
            return carry

        lax.fori_loop(0, NCH, body, 0)
        plsc.subcore_barrier()
        pltpu.sync_copy(acc_sh.at[stripe], out_hbm.at[c].at[stripe])

    return k


def _pair_gather_sc():
    mesh = plsc.VectorSubcoreMesh(core_axis_name="c", subcore_axis_name="s")

    @functools.partial(
        pl.kernel,
        out_type=jax.ShapeDtypeStruct((E, 128), jnp.float32),
        mesh=mesh,
        compiler_params=pltpu.CompilerParams(use_tc_tiling_on_sc=False),
        scratch_types=[
            pltpu.VMEM((NCH, CH), jnp.int32),
            pltpu.VMEM((NCH, CH), jnp.int32),
            pltpu.VMEM((CH, 128), jnp.float32),
            pltpu.SemaphoreType.DMA,
        ],
    )
    def k(a_hbm, b_hbm, src_hbm, dst_hbm, out_hbm, src_v, dst_v, rows_v, sem):
        c = lax.axis_index("c")
        s = lax.axis_index("s")
        wid = c * NS + s
        pltpu.sync_copy(src_hbm.at[wid], src_v)
        pltpu.sync_copy(dst_hbm.at[wid], dst_v)

        def body(i, carry):
            pltpu.async_copy(a_hbm.at[src_v.at[i]], rows_v, sem).wait()
            pltpu.async_copy(b_hbm.at[dst_v.at[i]], rows_v, sem, add=True).wait()
            pltpu.sync_copy(rows_v, out_hbm.at[pl.ds(wid * EPW + i * CH, CH)])
            return carry

        lax.fori_loop(0, NCH, body, 0)

    return k


def _leaky(x):
    return jnp.where(x >= 0, x, 0.01 * x)


def _layer1_tc(p_ref, x_ref, w1_ref, b1_ref, g1_ref, be1_ref, h1_ref, cnt_ref):
    agg = p_ref[0, :, :128] + p_ref[1, :, :128] + x_ref[...]
    cnt = p_ref[0, :, 128:129] + p_ref[1, :, 128:129] + 1.0
    xa = agg / cnt
    t = jnp.dot(xa, w1_ref[...], preferred_element_type=jnp.float32) + b1_ref[...]
    m = jnp.mean(t, axis=0, keepdims=True)
    v = jnp.mean((t - m) * (t - m), axis=0, keepdims=True)
    tn = g1_ref[...] * (t - m) * lax.rsqrt(v + 1e-5) + be1_ref[...]
    h1_ref[...] = _leaky(tn)
    cnt_ref[...] = cnt


def _layer2_tc(p_ref, h1_ref, cnt_ref, w2_ref, b2_ref, g2_ref, be2_ref,
               wp1a_ref, wp1b_ref, h_ref, a_ref, bb_ref):
    agg = (p_ref[0] + p_ref[1] + h1_ref[...]) / cnt_ref[...]
    t = jnp.dot(agg, w2_ref[...], preferred_element_type=jnp.float32) + b2_ref[...]
    m = jnp.mean(t, axis=0, keepdims=True)
    v = jnp.mean((t - m) * (t - m), axis=0, keepdims=True)
    tn = g2_ref[...] * (t - m) * lax.rsqrt(v + 1e-5) + be2_ref[...]
    h = _leaky(tn)
    h_ref[...] = h
    a_ref[...] = jnp.dot(h, wp1a_ref[...], preferred_element_type=jnp.float32)
    bb_ref[...] = jnp.dot(h, wp1b_ref[...], preferred_element_type=jnp.float32)


def _edge_mlp_tc(g_ref, bp1_ref, wp2_ref, bp2_ref, wp3_ref, bp3_ref, out_ref):
    t1 = jnp.maximum(g_ref[...] + bp1_ref[...], 0.0)
    t2 = jnp.dot(t1, wp2_ref[...], preferred_element_type=jnp.float32) + bp2_ref[...]
    t2 = jnp.maximum(t2, 0.0)
    t3 = jnp.dot(t2, wp3_ref[...], preferred_element_type=jnp.float32) + bp3_ref[...]
    out_ref[...] = jax.nn.sigmoid(t3)


_EBLK = 2000


def kernel(x, edge_index, W1, b1, gamma1, beta1, W2, b2, gamma2, beta2,
           Wp1, bp1, Wp2, bp2, Wp3, bp3):
    src = edge_index[0].reshape(NW, NCH, CH)
    dst = edge_index[1].reshape(NW, NCH, CH)
    x_aug = jnp.concatenate([x, jnp.ones((N, 16), jnp.float32)], axis=1)
    zeros144 = jnp.zeros((N, 144), jnp.float32)
    zeros128 = jnp.zeros((N, 128), jnp.float32)

    p1 = _seg_sum_sc(144)(x_aug, src, dst, zeros144)

    h1, cnt = pl.pallas_call(
        _layer1_tc,
        out_shape=(
            jax.ShapeDtypeStruct((N, 128), jnp.float32),
            jax.ShapeDtypeStruct((N, 1), jnp.float32),
        ),
    )(p1, x, W1, b1.reshape(1, 128), gamma1.reshape(1, 128), beta1.reshape(1, 128))

    p2 = _seg_sum_sc(128)(h1, src, dst, zeros128)

    h, A, B = pl.pallas_call(
        _layer2_tc,
        out_shape=(
            jax.ShapeDtypeStruct((N, 256), jnp.float32),
            jax.ShapeDtypeStruct((N, 128), jnp.float32),
            jax.ShapeDtypeStruct((N, 128), jnp.float32),
        ),
    )(p2, h1, cnt, W2, b2.reshape(1, 256), gamma2.reshape(1, 256),
      beta2.reshape(1, 256), Wp1[:256], Wp1[256:])

    G = _pair_gather_sc()(A, B, src, dst)

    pred = pl.pallas_call(
        _edge_mlp_tc,
        grid=(E // _EBLK,),
        in_specs=[
            pl.BlockSpec((_EBLK, 128), lambda i: (i, 0)),
            pl.BlockSpec((1, 128), lambda i: (0, 0)),
            pl.BlockSpec((128, 64), lambda i: (0, 0)),
            pl.BlockSpec((1, 64), lambda i: (0, 0)),
            pl.BlockSpec((64, 1), lambda i: (0, 0)),
            pl.BlockSpec((1, 1), lambda i: (0, 0)),
        ],
        out_specs=pl.BlockSpec((_EBLK, 1), lambda i: (i, 0)),
        out_shape=jax.ShapeDtypeStruct((E, 1), jnp.float32),
    )(G, bp1.reshape(1, 128), Wp2, bp2.reshape(1, 64), Wp3, bp3.reshape(1, 1))

    return (h, pred)
